# initial kernel scaffold (unmeasured)
import jax
import jax.numpy as jnp
from jax import lax
from jax.experimental import pallas as pl
from jax.experimental.pallas import tpu as pltpu

B, S, H, D = 2, 512, 8, 64
SCALE = D ** -0.5


def kernel(Q, K, V):
    def body(q_ref, k_ref, v_ref, out_ref, k_other, v_other, send_sems, recv_sems):
        my_x = lax.axis_index("x")
        my_y = lax.axis_index("y")
        peer = (1 - my_x, my_y)

        barrier_sem = pltpu.get_barrier_semaphore()
        pl.semaphore_signal(
            barrier_sem, inc=1, device_id=peer,
            device_id_type=pl.DeviceIdType.MESH,
        )
        pl.semaphore_wait(barrier_sem, 1)

        rdma_k = pltpu.make_async_remote_copy(
            src_ref=k_ref, dst_ref=k_other,
            send_sem=send_sems.at[0], recv_sem=recv_sems.at[0],
            device_id=peer, device_id_type=pl.DeviceIdType.MESH,
        )
        rdma_v = pltpu.make_async_remote_copy(
            src_ref=v_ref, dst_ref=v_other,
            send_sem=send_sems.at[1], recv_sem=recv_sems.at[1],
            device_id=peer, device_id_type=pl.DeviceIdType.MESH,
        )
        rdma_k.start()
        rdma_v.start()
        rdma_k.wait()
        rdma_v.wait()

        for b in range(B):
            for h in range(H):
                q = q_ref[b, :, h, :]
                k1 = k_ref[b, :, h, :]
                k2 = k_other[b, :, h, :]
                v1 = v_ref[b, :, h, :]
                v2 = v_other[b, :, h, :]
                s1 = lax.dot_general(
                    q, k1, (((1,), (1,)), ((), ())),
                    preferred_element_type=jnp.float32,
                ) * SCALE
                s2 = lax.dot_general(
                    q, k2, (((1,), (1,)), ((), ())),
                    preferred_element_type=jnp.float32,
                ) * SCALE
                m = jnp.maximum(
                    s1.max(axis=-1, keepdims=True),
                    s2.max(axis=-1, keepdims=True),
                )
                p1 = jnp.exp(s1 - m)
                p2 = jnp.exp(s2 - m)
                denom = (
                    p1.sum(axis=-1, keepdims=True)
                    + p2.sum(axis=-1, keepdims=True)
                )
                o1 = lax.dot_general(
                    p1, v1, (((1,), (0,)), ((), ())),
                    preferred_element_type=jnp.float32,
                )
                o2 = lax.dot_general(
                    p2, v2, (((1,), (0,)), ((), ())),
                    preferred_element_type=jnp.float32,
                )
                out_ref[b, :, h, :] = (o1 + o2) / denom

    return pl.pallas_call(
        body,
        out_shape=jax.ShapeDtypeStruct((B, S, H, D), jnp.float32),
        in_specs=[pl.BlockSpec(memory_space=pltpu.VMEM)] * 3,
        out_specs=pl.BlockSpec(memory_space=pltpu.VMEM),
        scratch_shapes=[
            pltpu.VMEM((B, S, H, D), jnp.float32),
            pltpu.VMEM((B, S, H, D), jnp.float32),
            pltpu.SemaphoreType.DMA((2,)),
            pltpu.SemaphoreType.DMA((2,)),
        ],
        compiler_params=pltpu.CompilerParams(collective_id=0),
    )(Q, K, V)


# baseline (device time: 136209 ns/iter reference)
import jax
import jax.numpy as jnp
from jax import lax
from jax.experimental import pallas as pl
from jax.experimental.pallas import tpu as pltpu

B, S, H, D = 2, 512, 8, 64
SCALE = D ** -0.5


def kernel(Q, K, V):
    def body(q_ref, k_ref, v_ref, out_ref, k_other, v_other, send_sems, recv_sems):
        my_x = lax.axis_index("x")
        my_y = lax.axis_index("y")
        peer = (1 - my_x, my_y)

        barrier_sem = pltpu.get_barrier_semaphore()
        pl.semaphore_signal(
            barrier_sem, inc=1, device_id=peer,
            device_id_type=pl.DeviceIdType.MESH,
        )
        pl.semaphore_wait(barrier_sem, 1)

        rdma_k = pltpu.make_async_remote_copy(
            src_ref=k_ref, dst_ref=k_other,
            send_sem=send_sems.at[0], recv_sem=recv_sems.at[0],
            device_id=peer, device_id_type=pl.DeviceIdType.MESH,
        )
        rdma_v = pltpu.make_async_remote_copy(
            src_ref=v_ref, dst_ref=v_other,
            send_sem=send_sems.at[1], recv_sem=recv_sems.at[1],
            device_id=peer, device_id_type=pl.DeviceIdType.MESH,
        )
        rdma_k.start()
        rdma_v.start()
        rdma_k.wait()
        rdma_v.wait()

        for b in range(B):
            for h in range(H):
                q = q_ref[b, :, h, :]
                k1 = k_ref[b, :, h, :]
                k2 = k_other[b, :, h, :]
                v1 = v_ref[b, :, h, :]
                v2 = v_other[b, :, h, :]
                s1 = lax.dot_general(
                    q, k1, (((1,), (1,)), ((), ())),
                    preferred_element_type=jnp.float32,
                ) * SCALE
                s2 = lax.dot_general(
                    q, k2, (((1,), (1,)), ((), ())),
                    preferred_element_type=jnp.float32,
                ) * SCALE
                m = jnp.maximum(
                    s1.max(axis=-1, keepdims=True),
                    s2.max(axis=-1, keepdims=True),
                )
                p1 = jnp.exp(s1 - m)
                p2 = jnp.exp(s2 - m)
                denom = (
                    p1.sum(axis=-1, keepdims=True)
                    + p2.sum(axis=-1, keepdims=True)
                )
                o1 = lax.dot_general(
                    p1, v1, (((1,), (0,)), ((), ())),
                    preferred_element_type=jnp.float32,
                )
                o2 = lax.dot_general(
                    p2, v2, (((1,), (0,)), ((), ())),
                    preferred_element_type=jnp.float32,
                )
                out_ref[b, :, h, :] = (o1 + o2) / denom

    return pl.pallas_call(
        body,
        out_shape=jax.ShapeDtypeStruct((B, S, H, D), jnp.float32),
        in_specs=[pl.BlockSpec(memory_space=pltpu.VMEM)] * 3,
        out_specs=pl.BlockSpec(memory_space=pltpu.VMEM),
        scratch_shapes=[
            pltpu.VMEM((B, S, H, D), jnp.float32),
            pltpu.VMEM((B, S, H, D), jnp.float32),
            pltpu.SemaphoreType.DMA((2,)),
            pltpu.SemaphoreType.DMA((2,)),
        ],
        compiler_params=pltpu.CompilerParams(
            collective_id=0, vmem_limit_bytes=100 * 1024 * 1024,
        ),
    )(Q, K, V)
